# baseline (speedup 1.0000x reference)
import jax
import jax.numpy as jnp
from jax.experimental import pallas as pl
from jax.experimental.pallas import tpu as pltpu

_SCORE_THR = 0.1
_BLOCK_I = 2048


def _amce_block(xt_ref, lab_ref, out_ref):
    xt = xt_ref[...]
    lab = lab_ref[0]
    m = jnp.max(xt, axis=0, keepdims=True)
    thr = jax.nn.sigmoid(m) - _SCORE_THR
    trow = jnp.where(thr > 0.0, jnp.log(thr) - jnp.log1p(-thr), -jnp.inf)
    cls = jax.lax.broadcasted_iota(jnp.int32, xt.shape, 0)
    is_lab = cls == lab
    a = jnp.abs(xt)
    t = jnp.exp2(a * jnp.float32(-1.4426950408889634))
    bce0 = jnp.maximum(xt, 0.0) + jnp.float32(0.6931471805599453) * jnp.log2(1.0 + t)
    w = (xt > trow) | is_lab
    contrib = jnp.where(w, bce0, 0.0) - jnp.where(is_lab, xt, 0.0)
    part = jnp.sum(contrib)

    @pl.when(pl.program_id(0) == 0)
    def _init():
        out_ref[0, 0] = 0.0

    out_ref[0, 0] += part


def kernel(cls_logits, labels):
    n_i, n_c = cls_logits.shape
    xt = cls_logits.T
    grid = n_i // _BLOCK_I
    labs = labels.reshape(grid, 1, _BLOCK_I)
    out = pl.pallas_call(
        _amce_block,
        grid=(grid,),
        in_specs=[
            pl.BlockSpec((n_c, _BLOCK_I), lambda i: (0, i)),
            pl.BlockSpec((1, 1, _BLOCK_I), lambda i: (i, 0, 0)),
        ],
        out_specs=pl.BlockSpec((1, 1), lambda i: (0, 0),
                               memory_space=pltpu.SMEM),
        out_shape=jax.ShapeDtypeStruct((1, 1), jnp.float32),
    )(xt, labs)
    return out[0, 0] / jnp.float32(n_i)


# speedup vs baseline: 1.3122x; 1.3122x over previous
import jax
import jax.numpy as jnp
from jax.experimental import pallas as pl
from jax.experimental.pallas import tpu as pltpu

_SCORE_THR = 0.1
_BLOCK_I = 1024


def _amce_block(xt_ref, lab_ref, out_ref):
    xt = xt_ref[...]
    lab = lab_ref[0]
    m = jnp.max(xt, axis=0, keepdims=True)
    thr = jax.nn.sigmoid(m) - _SCORE_THR
    trow = jnp.where(thr > 0.0, jnp.log(thr) - jnp.log1p(-thr), -jnp.inf)
    cls = jax.lax.broadcasted_iota(jnp.int32, xt.shape, 0)
    is_lab = cls == lab
    a = jnp.abs(xt)
    t = jnp.exp2(a * jnp.float32(-1.4426950408889634))
    bce0 = jnp.maximum(xt, 0.0) + jnp.float32(0.6931471805599453) * jnp.log2(1.0 + t)
    w = (xt > trow) | is_lab
    contrib = jnp.where(w, bce0, 0.0) - jnp.where(is_lab, xt, 0.0)
    ones = jnp.ones((1, contrib.shape[0]), jnp.float32)
    part = jnp.sum(jnp.dot(ones, contrib, preferred_element_type=jnp.float32))

    @pl.when(pl.program_id(0) == 0)
    def _init():
        out_ref[0, 0] = 0.0

    out_ref[0, 0] += part


def kernel(cls_logits, labels):
    n_i, n_c = cls_logits.shape
    xt = cls_logits.T
    grid = n_i // _BLOCK_I
    labs = labels.reshape(grid, 1, _BLOCK_I)
    out = pl.pallas_call(
        _amce_block,
        grid=(grid,),
        in_specs=[
            pl.BlockSpec((n_c, _BLOCK_I), lambda i: (0, i)),
            pl.BlockSpec((1, 1, _BLOCK_I), lambda i: (i, 0, 0)),
        ],
        out_specs=pl.BlockSpec((1, 1), lambda i: (0, 0),
                               memory_space=pltpu.SMEM),
        out_shape=jax.ShapeDtypeStruct((1, 1), jnp.float32),
    )(xt, labs)
    return out[0, 0] / jnp.float32(n_i)


# speedup vs baseline: 1.3515x; 1.0300x over previous
import jax
import jax.numpy as jnp
from jax.experimental import pallas as pl
from jax.experimental.pallas import tpu as pltpu

_SCORE_THR = 0.1
_BLOCK_I = 2048


def _amce_block(xt_ref, lab_ref, out_ref):
    xt = xt_ref[...]
    lab = lab_ref[0]
    m = jnp.max(xt, axis=0, keepdims=True)
    thr = jax.nn.sigmoid(m) - _SCORE_THR
    trow = jnp.where(thr > 0.0, jnp.log(thr) - jnp.log1p(-thr), -jnp.inf)
    cls = jax.lax.broadcasted_iota(jnp.int32, xt.shape, 0)
    is_lab = cls == lab
    a = jnp.abs(xt)
    t = jnp.exp2(a * jnp.float32(-1.4426950408889634))
    bce0 = jnp.maximum(xt, 0.0) + jnp.float32(0.6931471805599453) * jnp.log2(1.0 + t)
    w = (xt > trow) | is_lab
    contrib = jnp.where(w, bce0, 0.0) - jnp.where(is_lab, xt, 0.0)
    ones = jnp.ones((1, contrib.shape[0]), jnp.float32)
    part = jnp.sum(jnp.dot(ones, contrib, preferred_element_type=jnp.float32))

    @pl.when(pl.program_id(0) == 0)
    def _init():
        out_ref[0, 0] = 0.0

    out_ref[0, 0] += part


def kernel(cls_logits, labels):
    n_i, n_c = cls_logits.shape
    xt = cls_logits.T
    grid = n_i // _BLOCK_I
    labs = labels.reshape(grid, 1, _BLOCK_I)
    out = pl.pallas_call(
        _amce_block,
        grid=(grid,),
        in_specs=[
            pl.BlockSpec((n_c, _BLOCK_I), lambda i: (0, i)),
            pl.BlockSpec((1, 1, _BLOCK_I), lambda i: (i, 0, 0)),
        ],
        out_specs=pl.BlockSpec((1, 1), lambda i: (0, 0),
                               memory_space=pltpu.SMEM),
        out_shape=jax.ShapeDtypeStruct((1, 1), jnp.float32),
    )(xt, labs)
    return out[0, 0] / jnp.float32(n_i)
